# Initial kernel scaffold; baseline (speedup 1.0000x reference)
import jax
import jax.numpy as jnp
from jax.experimental import pallas as pl


def kernel(observed_trajectory_target, observed_trajectory_others, ln_g, ln_b, W_ih, W_hh, b_ih, b_hh, W_nb, b_nb, W_gat, a_src, a_dst, b_gat, Wq, bq, Wk, bk, Wv, bv, Wo, bo, W1, b1, W2, b2):
    raise NotImplementedError("write your pallas kernel here")


# speedup vs baseline: 1.4438x; 1.4438x over previous
import jax
import jax.numpy as jnp
from jax.experimental import pallas as pl
from jax.experimental.pallas import tpu as pltpu

_F32 = jnp.float32


def _body(xt_ref, oth_ref, lng_ref, lnb_ref, wih_ref, whh_ref, bih_ref,
          bhh_ref, wnb_ref, bnb_ref, wgatt_ref, wgat_ref, asrc_ref, adst_ref,
          bgat_ref, wq_ref, bq_ref, wk_ref, bk_ref, wv_ref, bv_ref, wo_ref,
          bo_ref, w1_ref, b1_ref, w2_ref, b2_ref, out_ref, xw_ref, hoth_ref):
    T, B, F = xt_ref.shape
    N = oth_ref.shape[0]
    H = whh_ref.shape[0]
    TB = T * B

    lng = lng_ref[:]
    lnb = lnb_ref[:]

    def ln(x):
        mu = jnp.mean(x, axis=-1, keepdims=True)
        xc = x - mu
        var = jnp.mean(xc * xc, axis=-1, keepdims=True)
        return xc * jax.lax.rsqrt(var + 1e-5) * lng + lnb

    def leaky(x):
        return jnp.where(x >= 0, x, 0.2 * x)

    def dot(a, b):
        return jnp.dot(a, b, preferred_element_type=_F32)

    x = ln(xt_ref[:].reshape(TB, F))
    xw_ref[:] = dot(x, wih_ref[:]) + bih_ref[:] + bhh_ref[:]

    whh = whh_ref[:]
    h = jnp.zeros((B, H), _F32)
    c = jnp.zeros((B, H), _F32)
    hs = []
    for t in range(T):
        g4 = xw_ref[t * B:(t + 1) * B, :] + dot(h, whh)
        i_g = jax.nn.sigmoid(g4[:, 0:H])
        f_g = jax.nn.sigmoid(g4[:, H:2 * H])
        g_g = jnp.tanh(g4[:, 2 * H:3 * H])
        o_g = jax.nn.sigmoid(g4[:, 3 * H:4 * H])
        c = f_g * c + i_g * g_g
        h = o_g * jnp.tanh(c)
        hs.append(h)
    lstm = jnp.concatenate(hs, axis=0)

    gt = dot(lstm, wgatt_ref[:])
    adst0 = jnp.sum(gt * adst_ref[:], axis=1, keepdims=True)
    e_self = leaky(jnp.sum(gt * asrc_ref[:], axis=1, keepdims=True) + adst0)

    on = ln(oth_ref[:].reshape(N * TB, F))
    hoth_ref[:] = jax.nn.relu(dot(on, wnb_ref[:]) + bnb_ref[:])

    w_row = dot(asrc_ref[:], wgat_ref[:])

    es = []
    for n in range(N):
        hn = hoth_ref[n * TB:(n + 1) * TB, :]
        s_n = jnp.sum(hn * w_row, axis=1, keepdims=True)
        es.append(leaky(s_n + adst0))
    es.append(e_self)
    e = jnp.concatenate(es, axis=1)
    m = jnp.max(e, axis=1, keepdims=True)
    p = jnp.exp(e - m)
    aw = p / jnp.sum(p, axis=1, keepdims=True)

    acc = jnp.zeros((TB, H), _F32)
    for n in range(N):
        acc = acc + aw[:, n:n + 1] * hoth_ref[n * TB:(n + 1) * TB, :]
    combined = dot(acc, wgatt_ref[:]) + aw[:, N:N + 1] * gt + bgat_ref[:]

    q = dot(combined[(T - 1) * B:TB, :], wq_ref[:]) + bq_ref[:]
    k = dot(combined, wk_ref[:]) + bk_ref[:]
    v = dot(combined, wv_ref[:]) + bv_ref[:]
    inv_scale = 1.0 / jnp.sqrt(jnp.asarray(H, _F32))
    scs = []
    for t in range(T):
        kt = k[t * B:(t + 1) * B, :]
        scs.append(jnp.sum(q * kt, axis=1, keepdims=True) * inv_scale)
    sc = jnp.concatenate(scs, axis=1)
    mt = jnp.max(sc, axis=1, keepdims=True)
    pt = jnp.exp(sc - mt)
    wt = pt / jnp.sum(pt, axis=1, keepdims=True)
    att = jnp.zeros((B, H), _F32)
    for t in range(T):
        att = att + wt[:, t:t + 1] * v[t * B:(t + 1) * B, :]
    att = dot(att, wo_ref[:]) + bo_ref[:]

    hid = jax.nn.relu(dot(att, w1_ref[:]) + b1_ref[:])
    out_ref[:] = dot(hid, w2_ref[:]) + b2_ref[:]


def kernel(observed_trajectory_target, observed_trajectory_others, ln_g, ln_b,
           W_ih, W_hh, b_ih, b_hh, W_nb, b_nb, W_gat, a_src, a_dst, b_gat,
           Wq, bq, Wk, bk, Wv, bv, Wo, bo, W1, b1, W2, b2):
    B, T, F = observed_trajectory_target.shape
    N = observed_trajectory_others.shape[2]
    H = W_hh.shape[1]

    xt = jnp.transpose(observed_trajectory_target, (1, 0, 2))
    otht = jnp.transpose(observed_trajectory_others, (2, 1, 0, 3))

    row = lambda v: v.reshape(1, -1)

    return pl.pallas_call(
        _body,
        out_shape=jax.ShapeDtypeStruct((B, 2), _F32),
        scratch_shapes=[
            pltpu.VMEM((T * B, 4 * H), _F32),
            pltpu.VMEM((N * T * B, H), _F32),
        ],
    )(xt, otht, row(ln_g), row(ln_b), W_ih.T, W_hh.T, row(b_ih), row(b_hh),
      W_nb.T, row(b_nb), W_gat.T, W_gat, row(a_src), row(a_dst), row(b_gat),
      Wq.T, row(bq), Wk.T, row(bk), Wv.T, row(bv), Wo.T, row(bo),
      W1.T, row(b1), W2.T, row(b2))
